# baseline (speedup 1.0000x reference)
import functools
import math

import jax
import jax.numpy as jnp
from jax import lax
from jax.experimental import pallas as pl
from jax.experimental.pallas import tpu as pltpu
from jax.experimental.pallas import tpu_sc as plsc

SEQ_LEN = 2048
BATCH = 64
DIM = 64
NC = 2
NS = 16
NW = NC * NS
ROWS = SEQ_LEN * BATCH
ROWS_W = ROWS // NW
SEQ_W = SEQ_LEN // NW
CHUNK_S = 2
CHUNK_R = CHUNK_S * BATCH
N_CHUNKS = SEQ_W // CHUNK_S
SCALE = math.sqrt(DIM)
LANES = 16
VPR = DIM // LANES

@functools.cache
def _build_kernel():
    mesh = plsc.VectorSubcoreMesh(
        core_axis_name="c", subcore_axis_name="s", num_cores=NC, num_subcores=NS
    )
    return pl.kernel(
        _emb_body,
        out_type=jax.ShapeDtypeStruct((ROWS, DIM), jnp.float32),
        mesh=mesh,
        scratch_types=[
            pltpu.VMEM((ROWS_W,), jnp.int32),
            pltpu.VMEM((SEQ_W * DIM,), jnp.float32),
            pltpu.VMEM((CHUNK_R, DIM), jnp.float32),
            pltpu.SemaphoreType.DMA,
        ],
        compiler_params=pltpu.CompilerParams(use_tc_tiling_on_sc=False),
    )


def _emb_body(idx_hbm, table_hbm, pe_hbm, out_hbm, idx_v, pe_v, buf, sem):
    wid = lax.axis_index("s") * NC + lax.axis_index("c")
    base = wid * ROWS_W

    pltpu.sync_copy(idx_hbm.at[pl.ds(base, ROWS_W)], idx_v)
    pltpu.sync_copy(pe_hbm.at[pl.ds(wid * SEQ_W * DIM, SEQ_W * DIM)], pe_v)

    def chunk_body(g, carry):
        idx_slice = idx_v.at[pl.ds(g * CHUNK_R, CHUNK_R)]
        pltpu.async_copy(table_hbm.at[idx_slice], buf, sem).wait()

        for sp in range(CHUNK_S):
            srow = g * CHUNK_S + sp
            pe_regs = [
                pe_v[pl.ds(srow * DIM + j * LANES, LANES)] for j in range(VPR)
            ]

            def row_body(r, c, pe_regs=pe_regs, sp=sp):
                row = sp * BATCH + r
                for j in range(VPR):
                    v = buf[row, pl.ds(j * LANES, LANES)]
                    buf[row, pl.ds(j * LANES, LANES)] = v * SCALE + pe_regs[j]
                return c

            lax.fori_loop(0, BATCH, row_body, 0, unroll=2)

        pltpu.sync_copy(buf, out_hbm.at[pl.ds(base + g * CHUNK_R, CHUNK_R)])
        return carry

    lax.fori_loop(0, N_CHUNKS, chunk_body, 0)


def kernel(source, word_table, pe):
    idx = source.reshape(ROWS)
    pe_flat = pe[:SEQ_LEN, 0, :].reshape(SEQ_LEN * DIM)
    out = _build_kernel()(idx, word_table, pe_flat)
    return out.reshape(SEQ_LEN, BATCH, DIM)


# speedup vs baseline: 1.1507x; 1.1507x over previous
import functools
import math

import jax
import jax.numpy as jnp
from jax import lax
from jax.experimental import pallas as pl
from jax.experimental.pallas import tpu as pltpu
from jax.experimental.pallas import tpu_sc as plsc

SEQ_LEN = 2048
BATCH = 64
DIM = 64
NC = 2
NS = 16
NW = NC * NS
ROWS = SEQ_LEN * BATCH
ROWS_W = ROWS // NW
SEQ_W = SEQ_LEN // NW
CHUNK_S = 2
CHUNK_R = CHUNK_S * BATCH
N_CHUNKS = SEQ_W // CHUNK_S
SCALE = math.sqrt(DIM)
LANES = 16
VPR = DIM // LANES

N_SLOTS = 4
LOOKAHEAD = 2


@functools.cache
def _build_kernel():
    mesh = plsc.VectorSubcoreMesh(
        core_axis_name="c", subcore_axis_name="s", num_cores=NC, num_subcores=NS
    )
    return pl.kernel(
        _emb_body,
        out_type=jax.ShapeDtypeStruct((ROWS, DIM), jnp.float32),
        mesh=mesh,
        scratch_types=[
            pltpu.VMEM((ROWS_W,), jnp.int32),
            pltpu.VMEM((SEQ_W * DIM,), jnp.float32),
            pltpu.VMEM((N_SLOTS, CHUNK_R, DIM), jnp.float32),
            [pltpu.SemaphoreType.DMA] * N_SLOTS,
            [pltpu.SemaphoreType.DMA] * N_SLOTS,
        ],
        compiler_params=pltpu.CompilerParams(use_tc_tiling_on_sc=False),
    )


def _emb_body(idx_hbm, table_hbm, pe_hbm, out_hbm, idx_v, pe_v, bufs, gsems, osems):
    wid = lax.axis_index("s") * NC + lax.axis_index("c")
    base = wid * ROWS_W

    pltpu.sync_copy(idx_hbm.at[pl.ds(base, ROWS_W)], idx_v)
    pltpu.sync_copy(pe_hbm.at[pl.ds(wid * SEQ_W * DIM, SEQ_W * DIM)], pe_v)

    def start_gather(g):
        slot = g % N_SLOTS
        idx_slice = idx_v.at[pl.ds(g * CHUNK_R, CHUNK_R)]
        return pltpu.async_copy(table_hbm.at[idx_slice], bufs.at[slot], gsems[slot])

    def start_out(g):
        slot = g % N_SLOTS
        return pltpu.async_copy(
            bufs.at[slot], out_hbm.at[pl.ds(base + g * CHUNK_R, CHUNK_R)], osems[slot]
        )

    gd = {}
    od = {}
    for g in range(LOOKAHEAD):
        gd[g] = start_gather(g)

    for g in range(N_CHUNKS):
        h = g + LOOKAHEAD
        if h < N_CHUNKS:
            prev = h - N_SLOTS
            if prev >= 0:
                od.pop(prev).wait()
            gd[h] = start_gather(h)

        gd.pop(g).wait()

        slot = g % N_SLOTS
        for sp in range(CHUNK_S):
            srow = g * CHUNK_S + sp
            pe_regs = [
                pe_v[pl.ds(srow * DIM + j * LANES, LANES)] for j in range(VPR)
            ]

            def row_body(r, c, pe_regs=pe_regs, sp=sp, slot=slot):
                row = sp * BATCH + r
                for j in range(VPR):
                    v = bufs[slot, row, pl.ds(j * LANES, LANES)]
                    bufs[slot, row, pl.ds(j * LANES, LANES)] = v * SCALE + pe_regs[j]
                return c

            lax.fori_loop(0, BATCH, row_body, 0, unroll=2)

        od[g] = start_out(g)

    for g in sorted(od):
        od.pop(g).wait()


def kernel(source, word_table, pe):
    idx = source.reshape(ROWS)
    pe_flat = pe[:SEQ_LEN, 0, :].reshape(SEQ_LEN * DIM)
    out = _build_kernel()(idx, word_table, pe_flat)
    return out.reshape(SEQ_LEN, BATCH, DIM)


# speedup vs baseline: 1.5589x; 1.3548x over previous
import functools
import math

import jax
import jax.numpy as jnp
from jax import lax
from jax.experimental import pallas as pl
from jax.experimental.pallas import tpu as pltpu
from jax.experimental.pallas import tpu_sc as plsc

SEQ_LEN = 2048
BATCH = 64
DIM = 64
NC = 2
NS = 16
NW = NC * NS
ROWS = SEQ_LEN * BATCH
ROWS_W = ROWS // NW
SEQ_W = SEQ_LEN // NW
CHUNK_S = 2
CHUNK_R = CHUNK_S * BATCH
N_CHUNKS = SEQ_W // CHUNK_S
SCALE = math.sqrt(DIM)
LANES = 16
VPR = DIM // LANES

N_SLOTS = 4
LOOKAHEAD = 2


@functools.cache
def _build_kernel():
    mesh = plsc.VectorSubcoreMesh(
        core_axis_name="c", subcore_axis_name="s", num_cores=NC, num_subcores=NS
    )
    return pl.kernel(
        _emb_body,
        out_type=jax.ShapeDtypeStruct((ROWS, 128), jnp.float32),
        mesh=mesh,
        scratch_types=[
            pltpu.VMEM((ROWS_W,), jnp.int32),
            pltpu.VMEM((SEQ_W * DIM,), jnp.float32),
            pltpu.VMEM((N_SLOTS, CHUNK_R, DIM), jnp.float32),
            [pltpu.SemaphoreType.DMA] * N_SLOTS,
            [pltpu.SemaphoreType.DMA] * N_SLOTS,
        ],
        compiler_params=pltpu.CompilerParams(use_tc_tiling_on_sc=False),
    )


def _emb_body(idx_hbm, table_hbm, pe_hbm, out_hbm, idx_v, pe_v, bufs, gsems, osems):
    wid = lax.axis_index("s") * NC + lax.axis_index("c")
    base = wid * ROWS_W

    pltpu.sync_copy(idx_hbm.at[pl.ds(base, ROWS_W)], idx_v)
    pltpu.sync_copy(pe_hbm.at[pl.ds(wid * SEQ_W * DIM, SEQ_W * DIM)], pe_v)

    def start_gather(g):
        slot = g % N_SLOTS
        idx_slice = idx_v.at[pl.ds(g * CHUNK_R, CHUNK_R)]
        return pltpu.async_copy(table_hbm.at[idx_slice], bufs.at[slot], gsems[slot])

    def start_out(g):
        slot = g % N_SLOTS
        return pltpu.async_copy(
            bufs.at[slot],
            out_hbm.at[pl.ds(base + g * CHUNK_R, CHUNK_R), pl.ds(0, DIM)],
            osems[slot],
        )

    gd = {}
    od = {}
    for g in range(LOOKAHEAD):
        gd[g] = start_gather(g)

    for g in range(N_CHUNKS):
        h = g + LOOKAHEAD
        if h < N_CHUNKS:
            prev = h - N_SLOTS
            if prev >= 0:
                od.pop(prev).wait()
            gd[h] = start_gather(h)

        gd.pop(g).wait()

        slot = g % N_SLOTS
        for sp in range(CHUNK_S):
            srow = g * CHUNK_S + sp
            pe_regs = [
                pe_v[pl.ds(srow * DIM + j * LANES, LANES)] for j in range(VPR)
            ]

            def row_body(r, c, pe_regs=pe_regs, sp=sp, slot=slot):
                row = sp * BATCH + r
                for j in range(VPR):
                    v = bufs[slot, row, pl.ds(j * LANES, LANES)]
                    bufs[slot, row, pl.ds(j * LANES, LANES)] = v * SCALE + pe_regs[j]
                return c

            lax.fori_loop(0, BATCH, row_body, 0, unroll=2)

        od[g] = start_out(g)

    for g in sorted(od):
        od.pop(g).wait()


def kernel(source, word_table, pe):
    idx = source.reshape(ROWS)
    pe_flat = pe[:SEQ_LEN, 0, :].reshape(SEQ_LEN * DIM)
    out = _build_kernel()(idx, word_table, pe_flat)
    return out.reshape(SEQ_LEN, BATCH, 128)[:, :, :DIM]
